# baseline (speedup 1.0000x reference)
import jax
import jax.numpy as jnp
from jax import lax
from jax.experimental import pallas as pl
from jax.experimental.pallas import tpu as pltpu
from jax.experimental.pallas import tpu_sc as plsc

B = 16384
F = 64
NUM_CORES = 2
NUM_SUBCORES = 16
NW = NUM_CORES * NUM_SUBCORES
BPW = B // NW

_PARAMS = pltpu.CompilerParams(
    needs_layout_passes=False, use_tc_tiling_on_sc=False)


def _gather_body(user_hbm, uf_hbm, ub_hbm, rows_hbm, ubg_hbm,
                 idx_u, uf_v, ub_v, sem_u, sem_ub):
    wid = lax.axis_index("s") * NUM_CORES + lax.axis_index("c")
    base = wid * BPW

    pltpu.sync_copy(user_hbm.at[pl.ds(base, BPW)], idx_u)
    cu = pltpu.async_copy(uf_hbm.at[idx_u], uf_v, sem_u)
    cub = pltpu.async_copy(ub_hbm.at[idx_u], ub_v, sem_ub)
    cu.wait()
    cub.wait()
    pltpu.sync_copy(uf_v, rows_hbm.at[pl.ds(base, BPW), :])
    pltpu.sync_copy(ub_v, ubg_hbm.at[pl.ds(base, BPW)])


def _dot_body(item_hbm, if_hbm, ib_hbm, rows_hbm, ubg_hbm, out_hbm,
              idx_i, uf_v, if_v, ub_v, ib_v, out_v, tr_v,
              sem_i, sem_ib, sem_r, sem_ub):
    wid = lax.axis_index("s") * NUM_CORES + lax.axis_index("c")
    base = wid * BPW

    pltpu.sync_copy(item_hbm.at[pl.ds(base, BPW)], idx_i)
    ci = pltpu.async_copy(if_hbm.at[idx_i], if_v, sem_i)
    cib = pltpu.async_copy(ib_hbm.at[idx_i], ib_v, sem_ib)
    cr = pltpu.async_copy(rows_hbm.at[pl.ds(base, BPW), :], uf_v, sem_r)
    cub = pltpu.async_copy(ubg_hbm.at[pl.ds(base, BPW)], ub_v, sem_ub)
    ci.wait()
    cib.wait()
    cr.wait()
    cub.wait()

    col16 = lax.iota(jnp.int32, 16) * 16

    def group(g, carry):
        e0 = g * 16
        for j in range(16):
            r = e0 + j
            acc = uf_v[r, pl.ds(0, 16)] * if_v[r, pl.ds(0, 16)]
            for k in range(1, 4):
                acc = acc + (uf_v[r, pl.ds(k * 16, 16)]
                             * if_v[r, pl.ds(k * 16, 16)])
            plsc.store_scatter(tr_v, [col16 + j], acc)
        tot = tr_v[pl.ds(0, 16)]
        for j in range(1, 16):
            tot = tot + tr_v[pl.ds(j * 16, 16)]
        tot = tot + ub_v[pl.ds(e0, 16)] + ib_v[pl.ds(e0, 16)]
        out_v[pl.ds(e0, 16)] = tot
        return carry

    lax.fori_loop(0, BPW // 16, group, 0)
    pltpu.sync_copy(out_v, out_hbm.at[pl.ds(base, BPW)])


def kernel(user, item, user_factors, item_factors, user_bias, item_bias):
    mesh = plsc.VectorSubcoreMesh(core_axis_name="c", subcore_axis_name="s")
    k1 = pl.kernel(
        _gather_body,
        out_type=(jax.ShapeDtypeStruct((B, F), jnp.float32),
                  jax.ShapeDtypeStruct((B,), jnp.float32)),
        mesh=mesh,
        compiler_params=_PARAMS,
        scratch_types=[
            pltpu.VMEM((BPW,), jnp.int32),
            pltpu.VMEM((BPW, F), jnp.float32),
            pltpu.VMEM((BPW,), jnp.float32),
            pltpu.SemaphoreType.DMA,
            pltpu.SemaphoreType.DMA,
        ],
    )
    k2 = pl.kernel(
        _dot_body,
        out_type=jax.ShapeDtypeStruct((B,), jnp.float32),
        mesh=mesh,
        compiler_params=_PARAMS,
        scratch_types=[
            pltpu.VMEM((BPW,), jnp.int32),
            pltpu.VMEM((BPW, F), jnp.float32),
            pltpu.VMEM((BPW, F), jnp.float32),
            pltpu.VMEM((BPW,), jnp.float32),
            pltpu.VMEM((BPW,), jnp.float32),
            pltpu.VMEM((BPW,), jnp.float32),
            pltpu.VMEM((256,), jnp.float32),
            pltpu.SemaphoreType.DMA,
            pltpu.SemaphoreType.DMA,
            pltpu.SemaphoreType.DMA,
            pltpu.SemaphoreType.DMA,
        ],
    )
    rows, ubg = k1(user.astype(jnp.int32), user_factors,
                   user_bias.reshape(-1))
    return k2(item.astype(jnp.int32), item_factors, item_bias.reshape(-1),
              rows, ubg)


# speedup vs baseline: 1.0315x; 1.0315x over previous
import jax
import jax.numpy as jnp
from jax import lax
from jax.experimental import pallas as pl
from jax.experimental.pallas import tpu as pltpu
from jax.experimental.pallas import tpu_sc as plsc

B = 16384
F = 64
NUM_CORES = 2
NUM_SUBCORES = 16
NW = NUM_CORES * NUM_SUBCORES
BPW = B // NW
HALF = BPW // 2


def _body(user_hbm, item_hbm, uf2_hbm, if2_hbm, ub_hbm, ib_hbm, out_hbm,
          idx_u, idx_i, idx2, uf_v, if_v, ub_v, ib_v, out_v, tr_v,
          sem_u, sem_i, sem_ub, sem_ib):
    wid = lax.axis_index("s") * NUM_CORES + lax.axis_index("c")
    base = wid * BPW

    pltpu.sync_copy(user_hbm.at[pl.ds(base, BPW)], idx_u)
    cub = pltpu.async_copy(ub_hbm.at[idx_u], ub_v, sem_ub)
    pltpu.sync_copy(item_hbm.at[pl.ds(base, BPW)], idx_i)
    cib = pltpu.async_copy(ib_hbm.at[idx_i], ib_v, sem_ib)

    col16 = lax.iota(jnp.int32, 16) * 16

    def half(h):
        h0 = h * HALF

        def shift(c, carry):
            s = h0 + c * 16
            idx2[pl.ds(c * 16, 16)] = lax.shift_right_logical(
                idx_u[pl.ds(s, 16)], 1)
            idx2[pl.ds(HALF + c * 16, 16)] = lax.shift_right_logical(
                idx_i[pl.ds(s, 16)], 1)
            return carry

        lax.fori_loop(0, HALF // 16, shift, 0)
        cu = pltpu.async_copy(uf2_hbm.at[idx2.at[pl.ds(0, HALF)]],
                              uf_v, sem_u)
        ci = pltpu.async_copy(if2_hbm.at[idx2.at[pl.ds(HALF, HALF)]],
                              if_v, sem_i)
        cu.wait()
        ci.wait()

        def group(g, carry):
            e0 = h0 + g * 16
            uvec = idx_u[pl.ds(e0, 16)]
            ivec = idx_i[pl.ds(e0, 16)]
            for j in range(16):
                r = g * 16 + j
                uoff = (uvec[j] & 1) * F
                ioff = (ivec[j] & 1) * F
                acc = (uf_v[r, pl.ds(uoff, 16)]
                       * if_v[r, pl.ds(ioff, 16)])
                for k in range(1, 4):
                    acc = acc + (uf_v[r, pl.ds(uoff + k * 16, 16)]
                                 * if_v[r, pl.ds(ioff + k * 16, 16)])
                plsc.store_scatter(tr_v, [col16 + j], acc)
            tot = tr_v[pl.ds(0, 16)]
            for j in range(1, 16):
                tot = tot + tr_v[pl.ds(j * 16, 16)]
            tot = tot + ub_v[pl.ds(e0, 16)] + ib_v[pl.ds(e0, 16)]
            out_v[pl.ds(e0, 16)] = tot
            return carry

        lax.fori_loop(0, HALF // 16, group, 0)

    half(0)
    half(1)
    cub.wait()
    cib.wait()
    pltpu.sync_copy(out_v, out_hbm.at[pl.ds(base, BPW)])


def kernel(user, item, user_factors, item_factors, user_bias, item_bias):
    mesh = plsc.VectorSubcoreMesh(core_axis_name="c", subcore_axis_name="s")
    k = pl.kernel(
        _body,
        out_type=jax.ShapeDtypeStruct((B,), jnp.float32),
        mesh=mesh,
        compiler_params=pltpu.CompilerParams(
            needs_layout_passes=False, use_tc_tiling_on_sc=True),
        scratch_types=[
            pltpu.VMEM((BPW,), jnp.int32),
            pltpu.VMEM((BPW,), jnp.int32),
            pltpu.VMEM((BPW,), jnp.int32),
            pltpu.VMEM((HALF, 2 * F), jnp.float32),
            pltpu.VMEM((HALF, 2 * F), jnp.float32),
            pltpu.VMEM((BPW,), jnp.float32),
            pltpu.VMEM((BPW,), jnp.float32),
            pltpu.VMEM((BPW,), jnp.float32),
            pltpu.VMEM((256,), jnp.float32),
            pltpu.SemaphoreType.DMA,
            pltpu.SemaphoreType.DMA,
            pltpu.SemaphoreType.DMA,
            pltpu.SemaphoreType.DMA,
        ],
    )
    uf2 = user_factors.reshape(500000, 2 * F)
    if2 = item_factors.reshape(500000, 2 * F)
    return k(user.astype(jnp.int32), item.astype(jnp.int32), uf2, if2,
             user_bias.reshape(-1), item_bias.reshape(-1))


# speedup vs baseline: 1.0374x; 1.0056x over previous
import jax
import jax.numpy as jnp
from jax import lax
from jax.experimental import pallas as pl
from jax.experimental.pallas import tpu as pltpu
from jax.experimental.pallas import tpu_sc as plsc

B = 16384
F = 64
NUM_CORES = 2
NUM_SUBCORES = 16
NW = NUM_CORES * NUM_SUBCORES
BPW = B // NW
GROUPS = BPW // 16


def _body(user_hbm, item_hbm, uf_hbm, if_hbm, ub_hbm, ib_hbm, out_hbm,
          idx_u, idx_i, uf_v, if_v, ub_v, ib_v, out_v, tr_v,
          sem_u, sem_i, sem_ub, sem_ib):
    wid = lax.axis_index("s") * NUM_CORES + lax.axis_index("c")
    base = wid * BPW

    pltpu.sync_copy(user_hbm.at[pl.ds(base, BPW)], idx_u)
    cu = pltpu.async_copy(uf_hbm.at[idx_u], uf_v, sem_u)
    cub = pltpu.async_copy(ub_hbm.at[idx_u], ub_v, sem_ub)
    pltpu.sync_copy(item_hbm.at[pl.ds(base, BPW)], idx_i)
    ci = pltpu.async_copy(if_hbm.at[idx_i], if_v, sem_i)
    cib = pltpu.async_copy(ib_hbm.at[idx_i], ib_v, sem_ib)
    cu.wait()
    ci.wait()
    cub.wait()
    cib.wait()

    col16 = lax.iota(jnp.int32, 16) * 16

    def group(g, carry):
        row0 = g * 16
        for e in range(16):
            r = row0 + e
            acc = uf_v[r, pl.ds(0, 16)] * if_v[r, pl.ds(0, 16)]
            for k in range(1, 4):
                acc = acc + uf_v[r, pl.ds(k * 16, 16)] * if_v[r, pl.ds(k * 16, 16)]
            plsc.store_scatter(tr_v, [col16 + e], acc)
        tot = tr_v[pl.ds(0, 16)]
        for j in range(1, 16):
            tot = tot + tr_v[pl.ds(j * 16, 16)]
        tot = tot + ub_v[pl.ds(row0, 16)] + ib_v[pl.ds(row0, 16)]
        out_v[pl.ds(row0, 16)] = tot
        return carry

    lax.fori_loop(0, GROUPS, group, 0)
    pltpu.sync_copy(out_v, out_hbm.at[pl.ds(base, BPW)])


def kernel(user, item, user_factors, item_factors, user_bias, item_bias):
    mesh = plsc.VectorSubcoreMesh(core_axis_name="c", subcore_axis_name="s")
    k = pl.kernel(
        _body,
        out_type=jax.ShapeDtypeStruct((B,), jnp.float32),
        mesh=mesh,
        compiler_params=pltpu.CompilerParams(
            needs_layout_passes=False, use_tc_tiling_on_sc=False),
        scratch_types=[
            pltpu.VMEM((BPW,), jnp.int32),
            pltpu.VMEM((BPW,), jnp.int32),
            pltpu.VMEM((BPW, F), jnp.float32),
            pltpu.VMEM((BPW, F), jnp.float32),
            pltpu.VMEM((BPW,), jnp.float32),
            pltpu.VMEM((BPW,), jnp.float32),
            pltpu.VMEM((BPW,), jnp.float32),
            pltpu.VMEM((256,), jnp.float32),
            pltpu.SemaphoreType.DMA,
            pltpu.SemaphoreType.DMA,
            pltpu.SemaphoreType.DMA,
            pltpu.SemaphoreType.DMA,
        ],
    )
    return k(user.astype(jnp.int32), item.astype(jnp.int32),
             user_factors, item_factors,
             user_bias.reshape(-1), item_bias.reshape(-1))
